# baseline (speedup 1.0000x reference)
import functools

import jax
import jax.numpy as jnp
from jax import lax
from jax.experimental import pallas as pl
from jax.experimental.pallas import tpu as pltpu
from jax.experimental.pallas import tpu_sc as plsc

MAXD = 4
L = 16
NC = 2
NS = 16
NW = NC * NS
CHUNK = 16384
NBUF = 2


def _sc_body(d_hbm, bias_hbm, out_hbm, table_v, din_v, dout_v, sin, sout):
    wid = lax.axis_index("s") * NC + lax.axis_index("c")
    n = d_hbm.shape[0]
    per_w = n // NW
    nch = per_w // CHUNK
    base = wid * per_w

    pltpu.sync_copy(bias_hbm, table_v)

    def copy_in(c, b):
        return pltpu.async_copy(
            d_hbm.at[pl.ds(base + c * CHUNK, CHUNK)], din_v[b], sin[b]
        )

    for c in range(min(NBUF, nch)):
        copy_in(c, c % NBUF)

    out_copies = {}
    for c in range(nch):
        b = c % NBUF
        pltpu.make_async_copy(
            d_hbm.at[pl.ds(base + c * CHUNK, CHUNK)], din_v[b], sin[b]
        ).wait()
        if c >= NBUF:
            out_copies.pop(c - NBUF).wait()

        @plsc.parallel_loop(0, CHUNK, step=L, unroll=16)
        def vec_body(i, b=b):
            idx = din_v[b][pl.ds(i, L)]
            idx = jnp.minimum(jnp.maximum(idx, 0), MAXD)
            dout_v[b][pl.ds(i, L)] = plsc.load_gather(table_v, [idx])
        out_copies[c] = pltpu.async_copy(
            dout_v[b], out_hbm.at[pl.ds(base + c * CHUNK, CHUNK)], sout[b]
        )
        if c + NBUF < nch:
            copy_in(c + NBUF, b)
    for c in sorted(out_copies):
        out_copies.pop(c).wait()


def kernel(distances, distance_bias):
    shape = distances.shape
    n = distances.size
    d_flat = distances.reshape(n)
    bias16 = jnp.zeros((L,), jnp.float32).at[: distance_bias.shape[0]].set(
        distance_bias
    )

    mesh = plsc.VectorSubcoreMesh(core_axis_name="c", subcore_axis_name="s")
    out = pl.kernel(
        _sc_body,
        mesh=mesh,
        compiler_params=pltpu.CompilerParams(needs_layout_passes=False),
        out_type=jax.ShapeDtypeStruct((n,), jnp.float32),
        scratch_types=[
            pltpu.VMEM((L,), jnp.float32),
            [pltpu.VMEM((CHUNK,), jnp.int32) for _ in range(NBUF)],
            [pltpu.VMEM((CHUNK,), jnp.float32) for _ in range(NBUF)],
            [pltpu.SemaphoreType.DMA for _ in range(NBUF)],
            [pltpu.SemaphoreType.DMA for _ in range(NBUF)],
        ],
    )(d_flat, bias16)
    return out.reshape(shape)


# speedup vs baseline: 1.0021x; 1.0021x over previous
import functools

import jax
import jax.numpy as jnp
from jax import lax
from jax.experimental import pallas as pl
from jax.experimental.pallas import tpu as pltpu
from jax.experimental.pallas import tpu_sc as plsc

MAXD = 4
L = 16
NC = 2
NS = 16
NW = NC * NS
CHUNK = 16384
NBUF = 2


def _sc_body(d_hbm, bias_hbm, out_hbm, table_v, din_v, dout_v, sin, sout):
    wid = lax.axis_index("s") * NC + lax.axis_index("c")
    n = d_hbm.shape[0]
    per_w = n // NW
    nch = per_w // CHUNK
    base = wid * per_w

    pltpu.sync_copy(bias_hbm, table_v)

    def copy_in(c, b):
        return pltpu.async_copy(
            d_hbm.at[pl.ds(base + c * CHUNK, CHUNK)], din_v[b], sin[b]
        )

    for c in range(min(NBUF, nch)):
        copy_in(c, c % NBUF)

    out_copies = {}
    for c in range(nch):
        b = c % NBUF
        pltpu.make_async_copy(
            d_hbm.at[pl.ds(base + c * CHUNK, CHUNK)], din_v[b], sin[b]
        ).wait()
        if c >= NBUF:
            out_copies.pop(c - NBUF).wait()

        @plsc.parallel_loop(0, CHUNK, step=L, unroll=8)
        def vec_body(i, b=b):
            idx = din_v[b][pl.ds(i, L)]
            idx = jnp.minimum(jnp.maximum(idx, 0), MAXD)
            dout_v[b][pl.ds(i, L)] = plsc.load_gather(table_v, [idx])
        out_copies[c] = pltpu.async_copy(
            dout_v[b], out_hbm.at[pl.ds(base + c * CHUNK, CHUNK)], sout[b]
        )
        if c + NBUF < nch:
            copy_in(c + NBUF, b)
    for c in sorted(out_copies):
        out_copies.pop(c).wait()


def kernel(distances, distance_bias):
    shape = distances.shape
    n = distances.size
    d_flat = distances.reshape(n)
    bias16 = jnp.zeros((L,), jnp.float32).at[: distance_bias.shape[0]].set(
        distance_bias
    )

    mesh = plsc.VectorSubcoreMesh(core_axis_name="c", subcore_axis_name="s")
    out = pl.kernel(
        _sc_body,
        mesh=mesh,
        compiler_params=pltpu.CompilerParams(needs_layout_passes=False),
        out_type=jax.ShapeDtypeStruct((n,), jnp.float32),
        scratch_types=[
            pltpu.VMEM((L,), jnp.float32),
            [pltpu.VMEM((CHUNK,), jnp.int32) for _ in range(NBUF)],
            [pltpu.VMEM((CHUNK,), jnp.float32) for _ in range(NBUF)],
            [pltpu.SemaphoreType.DMA for _ in range(NBUF)],
            [pltpu.SemaphoreType.DMA for _ in range(NBUF)],
        ],
    )(d_flat, bias16)
    return out.reshape(shape)


# speedup vs baseline: 2.4183x; 2.4131x over previous
import jax
import jax.numpy as jnp
from jax import lax
from jax.experimental import pallas as pl
from jax.experimental.pallas import tpu as pltpu
from jax.experimental.pallas import tpu_sc as plsc

MAXD = 4
L = 16
NC = 2
NS = 16
NW = NC * NS
COLS = 2048
CROWS = 8
NBUF = 2


def _sc_body(d3_hbm, bias_hbm, out3_hbm, table_v, din_v, dout_v, sin, sout):
    wid = lax.axis_index("s") * NC + lax.axis_index("c")
    n = d3_hbm.size
    rows = n // COLS
    d_hbm = d3_hbm.reshape(rows, COLS)
    out_hbm = out3_hbm.reshape(rows, COLS)
    rows_w = rows // NW
    nch = rows_w // CROWS
    base = wid * rows_w

    pltpu.sync_copy(bias_hbm, table_v)

    def copy_in(c, b):
        return pltpu.async_copy(
            d_hbm.at[pl.ds(base + c * CROWS, CROWS), :], din_v[b], sin[b]
        )

    for c in range(min(NBUF, nch)):
        copy_in(c, c % NBUF)

    out_copies = {}
    for c in range(nch):
        b = c % NBUF
        pltpu.make_async_copy(
            d_hbm.at[pl.ds(base + c * CROWS, CROWS), :], din_v[b], sin[b]
        ).wait()
        if c >= NBUF:
            out_copies.pop(c - NBUF).wait()

        @plsc.parallel_loop(0, COLS, step=L, unroll=1)
        def vec_body(i, b=b):
            for r in range(CROWS):
                idx = din_v[b][r, pl.ds(i, L)]
                idx = jnp.minimum(jnp.maximum(idx, 0), MAXD)
                dout_v[b][r, pl.ds(i, L)] = plsc.load_gather(table_v, [idx])

        out_copies[c] = pltpu.async_copy(
            dout_v[b], out_hbm.at[pl.ds(base + c * CROWS, CROWS), :], sout[b]
        )
        if c + NBUF < nch:
            copy_in(c + NBUF, b)
    for c in sorted(out_copies):
        out_copies.pop(c).wait()


def kernel(distances, distance_bias):
    shape = distances.shape
    bias16 = jnp.zeros((L,), jnp.float32).at[: distance_bias.shape[0]].set(
        distance_bias
    )

    mesh = plsc.VectorSubcoreMesh(core_axis_name="c", subcore_axis_name="s")
    out = pl.kernel(
        _sc_body,
        mesh=mesh,
        compiler_params=pltpu.CompilerParams(needs_layout_passes=False),
        out_type=jax.ShapeDtypeStruct(shape, jnp.float32),
        scratch_types=[
            pltpu.VMEM((L,), jnp.float32),
            [pltpu.VMEM((CROWS, COLS), jnp.int32) for _ in range(NBUF)],
            [pltpu.VMEM((CROWS, COLS), jnp.float32) for _ in range(NBUF)],
            [pltpu.SemaphoreType.DMA for _ in range(NBUF)],
            [pltpu.SemaphoreType.DMA for _ in range(NBUF)],
        ],
    )(distances, bias16)
    return out


# speedup vs baseline: 2.5632x; 1.0599x over previous
import jax
import jax.numpy as jnp
from jax import lax
from jax.experimental import pallas as pl
from jax.experimental.pallas import tpu as pltpu
from jax.experimental.pallas import tpu_sc as plsc

MAXD = 4
L = 16
NC = 2
NS = 16
NW = NC * NS
COLS = 2048
CROWS = 8
NBUF = 3


def _sc_body(d3_hbm, bias_hbm, out3_hbm, table_v, din_v, dout_v, sin, sout):
    wid = lax.axis_index("s") * NC + lax.axis_index("c")
    n = d3_hbm.size
    rows = n // COLS
    d_hbm = d3_hbm.reshape(rows, COLS)
    out_hbm = out3_hbm.reshape(rows, COLS)
    rows_w = rows // NW
    nch = rows_w // CROWS
    base = wid * rows_w

    pltpu.sync_copy(bias_hbm, table_v)

    def copy_in(c, b):
        return pltpu.async_copy(
            d_hbm.at[pl.ds(base + c * CROWS, CROWS), :], din_v[b], sin[b]
        )

    for c in range(min(NBUF, nch)):
        copy_in(c, c % NBUF)

    out_copies = {}
    for c in range(nch):
        b = c % NBUF
        pltpu.make_async_copy(
            d_hbm.at[pl.ds(base + c * CROWS, CROWS), :], din_v[b], sin[b]
        ).wait()
        if c >= NBUF:
            out_copies.pop(c - NBUF).wait()

        @plsc.parallel_loop(0, COLS, step=L, unroll=1)
        def vec_body(i, b=b):
            for r in range(CROWS):
                idx = din_v[b][r, pl.ds(i, L)]
                idx = jnp.minimum(jnp.maximum(idx, 0), MAXD)
                dout_v[b][r, pl.ds(i, L)] = plsc.load_gather(table_v, [idx])

        out_copies[c] = pltpu.async_copy(
            dout_v[b], out_hbm.at[pl.ds(base + c * CROWS, CROWS), :], sout[b]
        )
        if c + NBUF < nch:
            copy_in(c + NBUF, b)
    for c in sorted(out_copies):
        out_copies.pop(c).wait()


def kernel(distances, distance_bias):
    shape = distances.shape
    bias16 = jnp.zeros((L,), jnp.float32).at[: distance_bias.shape[0]].set(
        distance_bias
    )

    mesh = plsc.VectorSubcoreMesh(core_axis_name="c", subcore_axis_name="s")
    out = pl.kernel(
        _sc_body,
        mesh=mesh,
        compiler_params=pltpu.CompilerParams(needs_layout_passes=False),
        out_type=jax.ShapeDtypeStruct(shape, jnp.float32),
        scratch_types=[
            pltpu.VMEM((L,), jnp.float32),
            [pltpu.VMEM((CROWS, COLS), jnp.int32) for _ in range(NBUF)],
            [pltpu.VMEM((CROWS, COLS), jnp.float32) for _ in range(NBUF)],
            [pltpu.SemaphoreType.DMA for _ in range(NBUF)],
            [pltpu.SemaphoreType.DMA for _ in range(NBUF)],
        ],
    )(distances, bias16)
    return out
